# Initial kernel scaffold; baseline (speedup 1.0000x reference)
import jax
import jax.numpy as jnp
from jax.experimental import pallas as pl


def kernel(x):
    raise NotImplementedError("write your pallas kernel here")


# speedup vs baseline: 4.3030x; 4.3030x over previous
import functools

import jax
import jax.numpy as jnp
from jax import lax
from jax.experimental import pallas as pl
from jax.experimental.pallas import tpu as pltpu
from jax.experimental.pallas import tpu_sc as plsc

R = 64
C = 32768
K = 512
L = 16
NVR = C // L
NB = 256
NW = 32
ROWS_PER_W = R // NW


def _keys(v):
    b = lax.bitcast_convert_type(v + 0.0, jnp.int32)
    m = lax.shift_right_arithmetic(b, 31)
    return b ^ (m & jnp.int32(0x7FFFFFFF))


def _splat_pop(mask):
    return plsc.all_reduce_population_count(mask)


def _kmax_body(x_hbm, out_hbm, row_v, cmp_v, hist_v, tot_v, out_v):
    wid = lax.axis_index("s") * 2 + lax.axis_index("c")
    lane = lax.iota(jnp.int32, L)
    lane_base = lane * NB
    zeros16 = jnp.zeros((L,), jnp.int32)
    ones16 = jnp.ones((L,), jnp.int32)

    def radix_level(shift, is_top, prefix, kk, nn):
        def zbody(i, c):
            hist_v[pl.ds(i * L, L)] = zeros16
            return c

        lax.fori_loop(0, NB * L // L, zbody, 0)

        def hbody(i, c):
            v = row_v[pl.ds(i * L, L)]
            key = _keys(v)
            digit = lax.shift_right_arithmetic(key, shift) & jnp.int32(0xFF)
            if is_top:
                digit = digit ^ jnp.int32(0x80)
                plsc.addupdate_scatter(hist_v, [lane_base + digit], ones16)
            else:
                pm = lax.shift_right_arithmetic(key, shift + 8) == prefix
                plsc.addupdate_scatter(
                    hist_v, [lane_base + digit], ones16, mask=pm
                )
            return c

        lax.fori_loop(0, NVR, hbody, 0)

        nk = nn - kk

        def tbody(g, carry):
            cnt_vec, run_max, carry_p = carry
            tot = zeros16
            for l in range(L):
                tot = tot + hist_v[pl.ds(l * NB + g * L, L)]
            tot_v[pl.ds(g * L, L)] = tot
            p = plsc.cumsum(tot) + carry_p
            cond = p <= nk
            cnt_vec = cnt_vec + _splat_pop(cond)
            run_max = jnp.maximum(run_max, jnp.where(cond, p, 0))
            return cnt_vec, run_max, jnp.max(p)

        cnt_vec, run_max, _ = lax.fori_loop(
            0, NB // L, tbody, (zeros16, zeros16, jnp.int32(0))
        )
        dstar = jnp.max(cnt_vec)
        p_dm1 = jnp.max(run_max)

        def cbody(g, acc):
            tot = tot_v[pl.ds(g * L, L)]
            dv = g * L + lane
            return jnp.maximum(acc, jnp.where(dv == dstar, tot, 0))

        c_at = jnp.max(lax.fori_loop(0, NB // L, cbody, zeros16))

        p_at = p_dm1 + c_at
        kk_next = kk - (nn - p_at)
        draw = dstar ^ jnp.int32(0x80) if is_top else dstar
        if is_top:
            prefix_next = lax.shift_right_arithmetic(
                lax.shift_left(draw, 24), 24
            )
        else:
            prefix_next = lax.shift_left(prefix, 8) | draw
        return prefix_next, kk_next, c_at

    def do_row(r, c):
        row = wid * ROWS_PER_W + r
        pltpu.sync_copy(x_hbm.at[row], row_v)

        pre, kk, nn = jnp.int32(0), jnp.int32(K), jnp.int32(C)
        pre, kk, nn = radix_level(24, True, pre, kk, nn)
        pre, kk, nn = radix_level(16, False, pre, kk, nn)
        pre, kk, nn = radix_level(8, False, pre, kk, nn)
        tkey, need_eq, _ = radix_level(0, False, pre, kk, nn)

        def sbody(i, off_vec):
            v = row_v[pl.ds(i * L, L)]
            key = _keys(v)
            ge = key >= tkey
            gei = ge.astype(jnp.int32)
            rank = plsc.cumsum(gei) - gei
            plsc.store_scatter(cmp_v, [off_vec + rank], v, mask=ge)
            return off_vec + _splat_pop(ge)

        off_final = lax.fori_loop(0, NVR, sbody, zeros16)
        plsc.store_scatter(
            cmp_v, [off_final + lane], jnp.full((L,), -jnp.inf, jnp.float32)
        )
        count_ge = jnp.max(off_final)

        def mbody(i, carry):
            off_vec, eqc_vec = carry
            v = cmp_v[pl.ds(i * L, L)]
            key = _keys(v)
            eq = key == tkey
            gt = key > tkey
            eqi = eq.astype(jnp.int32)
            eqx = plsc.cumsum(eqi) - eqi
            sel = gt | (eq & ((eqc_vec + eqx) < need_eq))
            seli = sel.astype(jnp.int32)
            rank = plsc.cumsum(seli) - seli
            plsc.store_scatter(out_v, [off_vec + rank], v, mask=sel)
            return off_vec + _splat_pop(sel), eqc_vec + _splat_pop(eq)

        nvr2 = (count_ge + jnp.int32(L)) // jnp.int32(L)
        lax.fori_loop(0, nvr2, mbody, (zeros16, zeros16))
        pltpu.sync_copy(out_v.at[pl.ds(0, K)], out_hbm.at[row])
        return c

    lax.fori_loop(0, ROWS_PER_W, do_row, 0)


_mesh = plsc.VectorSubcoreMesh(core_axis_name="c", subcore_axis_name="s")

_kmax_sc = functools.partial(
    pl.kernel,
    out_type=jax.ShapeDtypeStruct((R, K), jnp.float32),
    mesh=_mesh,
    scratch_types=[
        pltpu.VMEM((C,), jnp.float32),
        pltpu.VMEM((C + L,), jnp.float32),
        pltpu.VMEM((NB * L,), jnp.int32),
        pltpu.VMEM((NB,), jnp.int32),
        pltpu.VMEM((K + L,), jnp.float32),
    ],
    compiler_params=pltpu.CompilerParams(needs_layout_passes=False),
)(_kmax_body)


@jax.jit
def kernel(x):
    return _kmax_sc(x)


# speedup vs baseline: 5.9600x; 1.3851x over previous
import functools

import jax
import jax.numpy as jnp
from jax import lax
from jax.experimental import pallas as pl
from jax.experimental.pallas import tpu as pltpu
from jax.experimental.pallas import tpu_sc as plsc

R = 64
C = 32768
K = 512
L = 16
NVR = C // L
NB = 256
NW = 32
ROWS_PER_W = R // NW
UNROLL = 8


def _keys(v):
    b = lax.bitcast_convert_type(v + 0.0, jnp.int32)
    m = lax.shift_right_arithmetic(b, 31)
    return b ^ (m & jnp.int32(0x7FFFFFFF))


def _splat_pop(mask):
    return plsc.all_reduce_population_count(mask)


def _kmax_body(x_hbm, out_hbm, row_v, cmp_v, side_v, hist_v, tot_v, out_v):
    wid = lax.axis_index("s") * 2 + lax.axis_index("c")
    lane = lax.iota(jnp.int32, L)
    lane_base = lane * NB
    zeros16 = jnp.zeros((L,), jnp.int32)
    ones16 = jnp.ones((L,), jnp.int32)
    neginf16 = jnp.full((L,), -jnp.inf, jnp.float32)

    def zero_hist():
        def zbody(i, c):
            hist_v[pl.ds(i * L, L)] = zeros16
            return c

        lax.fori_loop(0, NB, zbody, 0)

    def pick_digit(kk, nn):
        nk = nn - kk

        def tbody(g, carry):
            cnt_vec, run_max, carry_p = carry
            tot = zeros16
            for l in range(L):
                tot = tot + hist_v[pl.ds(l * NB + g * L, L)]
            tot_v[pl.ds(g * L, L)] = tot
            p = plsc.cumsum(tot) + carry_p
            cond = p <= nk
            cnt_vec = cnt_vec + _splat_pop(cond)
            run_max = jnp.maximum(run_max, jnp.where(cond, p, 0))
            return cnt_vec, run_max, jnp.max(p)

        cnt_vec, run_max, _ = lax.fori_loop(
            0, NB // L, tbody, (zeros16, zeros16, jnp.int32(0))
        )
        dstar = jnp.max(cnt_vec)
        p_dm1 = jnp.max(run_max)

        def cbody(g, acc):
            tot = tot_v[pl.ds(g * L, L)]
            dv = g * L + lane
            return jnp.maximum(acc, jnp.where(dv == dstar, tot, 0))

        c_at = jnp.max(lax.fori_loop(0, NB // L, cbody, zeros16))
        kk_next = kk - (nn - (p_dm1 + c_at))
        return dstar, kk_next, c_at

    def do_row(r, c):
        row = wid * ROWS_PER_W + r
        pltpu.sync_copy(x_hbm.at[row], row_v)

        zero_hist()

        def h0body(i, c):
            for u in range(UNROLL):
                v = row_v[pl.ds((i * UNROLL + u) * L, L)]
                key = _keys(v)
                digit = (
                    lax.shift_right_arithmetic(key, 24) & jnp.int32(0xFF)
                ) ^ jnp.int32(0x80)
                plsc.addupdate_scatter(hist_v, [lane_base + digit], ones16)
            return c

        lax.fori_loop(0, NVR // UNROLL, h0body, 0)
        d0, kk, nn = pick_digit(jnp.int32(K), jnp.int32(C))
        pre1 = lax.shift_right_arithmetic(
            lax.shift_left(d0 ^ jnp.int32(0x80), 24), 24
        )

        zero_hist()

        def h1body(i, off_vec):
            for u in range(UNROLL):
                v = row_v[pl.ds((i * UNROLL + u) * L, L)]
                key = _keys(v)
                pm = lax.shift_right_arithmetic(key, 24) == pre1
                digit = lax.shift_right_arithmetic(key, 16) & jnp.int32(0xFF)
                plsc.addupdate_scatter(
                    hist_v, [lane_base + digit], ones16, mask=pm
                )
                pmi = pm.astype(jnp.int32)
                rank = plsc.cumsum(pmi) - pmi
                plsc.store_scatter(side_v, [off_vec + rank], v, mask=pm)
                off_vec = off_vec + _splat_pop(pm)
            return off_vec

        side_off = lax.fori_loop(0, NVR // UNROLL, h1body, zeros16)
        plsc.store_scatter(side_v, [side_off + lane], neginf16)
        n_side = jnp.max(side_off)
        d1, kk, nn = pick_digit(kk, nn)
        pre2 = lax.shift_left(pre1, 8) | d1

        nvr_s = (n_side + jnp.int32(L)) // jnp.int32(L)

        def hs_factory(shift, prefix):
            def hsbody(i, c):
                v = side_v[pl.ds(i * L, L)]
                key = _keys(v)
                pm = lax.shift_right_arithmetic(key, shift + 8) == prefix
                digit = lax.shift_right_arithmetic(key, shift) & jnp.int32(
                    0xFF
                )
                plsc.addupdate_scatter(
                    hist_v, [lane_base + digit], ones16, mask=pm
                )
                return c

            return hsbody

        zero_hist()
        lax.fori_loop(0, nvr_s, hs_factory(8, pre2), 0)
        d2, kk, nn = pick_digit(kk, nn)
        pre3 = lax.shift_left(pre2, 8) | d2

        zero_hist()
        lax.fori_loop(0, nvr_s, hs_factory(0, pre3), 0)
        d3, need_eq, _ = pick_digit(kk, nn)
        tkey = lax.shift_left(pre3, 8) | d3

        def sbody(i, off_vec):
            for u in range(UNROLL):
                v = row_v[pl.ds((i * UNROLL + u) * L, L)]
                key = _keys(v)
                ge = key >= tkey
                gei = ge.astype(jnp.int32)
                rank = plsc.cumsum(gei) - gei
                plsc.store_scatter(cmp_v, [off_vec + rank], v, mask=ge)
                off_vec = off_vec + _splat_pop(ge)
            return off_vec

        off_final = lax.fori_loop(0, NVR // UNROLL, sbody, zeros16)
        plsc.store_scatter(cmp_v, [off_final + lane], neginf16)
        count_ge = jnp.max(off_final)

        def mbody(i, carry):
            off_vec, eqc_vec = carry
            v = cmp_v[pl.ds(i * L, L)]
            key = _keys(v)
            eq = key == tkey
            gt = key > tkey
            eqi = eq.astype(jnp.int32)
            eqx = plsc.cumsum(eqi) - eqi
            sel = gt | (eq & ((eqc_vec + eqx) < need_eq))
            seli = sel.astype(jnp.int32)
            rank = plsc.cumsum(seli) - seli
            plsc.store_scatter(out_v, [off_vec + rank], v, mask=sel)
            return off_vec + _splat_pop(sel), eqc_vec + _splat_pop(eq)

        nvr2 = (count_ge + jnp.int32(L)) // jnp.int32(L)
        lax.fori_loop(0, nvr2, mbody, (zeros16, zeros16))
        pltpu.sync_copy(out_v.at[pl.ds(0, K)], out_hbm.at[row])
        return c

    lax.fori_loop(0, ROWS_PER_W, do_row, 0)


_mesh = plsc.VectorSubcoreMesh(core_axis_name="c", subcore_axis_name="s")

_kmax_sc = functools.partial(
    pl.kernel,
    out_type=jax.ShapeDtypeStruct((R, K), jnp.float32),
    mesh=_mesh,
    scratch_types=[
        pltpu.VMEM((C,), jnp.float32),
        pltpu.VMEM((C + L,), jnp.float32),
        pltpu.VMEM((C + L,), jnp.float32),
        pltpu.VMEM((NB * L,), jnp.int32),
        pltpu.VMEM((NB,), jnp.int32),
        pltpu.VMEM((K + L,), jnp.float32),
    ],
    compiler_params=pltpu.CompilerParams(needs_layout_passes=False),
)(_kmax_body)


@jax.jit
def kernel(x):
    return _kmax_sc(x)


# speedup vs baseline: 18.8784x; 3.1675x over previous
import functools

import jax
import jax.numpy as jnp
from jax import lax
from jax.experimental import pallas as pl
from jax.experimental.pallas import tpu as pltpu
from jax.experimental.pallas import tpu_sc as plsc

R = 64
C = 32768
K = 512
L = 16
NVR = C // L
NB = 256
NW = 32
ROWS_PER_W = R // NW
UNROLL = 8


def _keys(v):
    b = lax.bitcast_convert_type(v + 0.0, jnp.int32)
    m = lax.shift_right_arithmetic(b, 31)
    return b ^ (m & jnp.int32(0x7FFFFFFF))


def _splat_pop(mask):
    return plsc.all_reduce_population_count(mask)


def _kmax_body(x_hbm, out_hbm, row_v, cmp_v, side_v, hist_v, tot_v, out_v):
    wid = lax.axis_index("s") * 2 + lax.axis_index("c")
    lane = lax.iota(jnp.int32, L)
    lane_base = lane * NB
    zeros16 = jnp.zeros((L,), jnp.int32)
    ones16 = jnp.ones((L,), jnp.int32)
    neginf16 = jnp.full((L,), -jnp.inf, jnp.float32)

    def zero_hist():
        @plsc.parallel_loop(0, NB, unroll=8)
        def _(i):
            hist_v[pl.ds(i * L, L)] = zeros16

    def pick_digit(kk, nn):
        nk = nn - kk

        def tbody(g, carry):
            cnt_vec, run_max, carry_p = carry
            tot = zeros16
            for l in range(L):
                tot = tot + hist_v[pl.ds(l * NB + g * L, L)]
            tot_v[pl.ds(g * L, L)] = tot
            p = plsc.cumsum(tot) + carry_p
            cond = p <= nk
            cnt_vec = cnt_vec + _splat_pop(cond)
            run_max = jnp.maximum(run_max, jnp.where(cond, p, 0))
            return cnt_vec, run_max, jnp.max(p)

        cnt_vec, run_max, _ = lax.fori_loop(
            0, NB // L, tbody, (zeros16, zeros16, jnp.int32(0))
        )
        dstar = jnp.max(cnt_vec)
        p_dm1 = jnp.max(run_max)

        def cbody(g, acc):
            tot = tot_v[pl.ds(g * L, L)]
            dv = g * L + lane
            return jnp.maximum(acc, jnp.where(dv == dstar, tot, 0))

        c_at = jnp.max(lax.fori_loop(0, NB // L, cbody, zeros16))
        kk_next = kk - (nn - (p_dm1 + c_at))
        return dstar, kk_next, c_at

    def do_row(r, c):
        row = wid * ROWS_PER_W + r
        pltpu.sync_copy(x_hbm.at[row], row_v)

        zero_hist()

        @plsc.parallel_loop(0, NVR, unroll=UNROLL)
        def _(i):
            key = _keys(row_v[pl.ds(i * L, L)])
            digit = (
                lax.shift_right_arithmetic(key, 24) & jnp.int32(0xFF)
            ) ^ jnp.int32(0x80)
            plsc.addupdate_scatter(hist_v, [lane_base + digit], ones16)

        d0, kk, nn = pick_digit(jnp.int32(K), jnp.int32(C))
        pre1 = lax.shift_right_arithmetic(
            lax.shift_left(d0 ^ jnp.int32(0x80), 24), 24
        )

        zero_hist()

        @plsc.parallel_loop(0, NVR, unroll=UNROLL, carry=zeros16)
        def side_off(i, off_vec):
            v = row_v[pl.ds(i * L, L)]
            key = _keys(v)
            pm = lax.shift_right_arithmetic(key, 24) == pre1
            digit = lax.shift_right_arithmetic(key, 16) & jnp.int32(0xFF)
            plsc.addupdate_scatter(
                hist_v, [lane_base + digit], ones16, mask=pm
            )
            pmi = pm.astype(jnp.int32)
            rank = plsc.cumsum(pmi) - pmi
            plsc.store_scatter(side_v, [off_vec + rank], v, mask=pm)
            return off_vec + _splat_pop(pm)

        plsc.store_scatter(side_v, [side_off + lane], neginf16)
        n_side = jnp.max(side_off)
        d1, kk, nn = pick_digit(kk, nn)
        pre2 = lax.shift_left(pre1, 8) | d1

        nvr_s = (n_side + jnp.int32(L)) // jnp.int32(L)

        def side_hist(shift, prefix):
            zero_hist()

            @plsc.parallel_loop(0, nvr_s, unroll=2)
            def _(i):
                key = _keys(side_v[pl.ds(i * L, L)])
                pm = lax.shift_right_arithmetic(key, shift + 8) == prefix
                digit = lax.shift_right_arithmetic(key, shift) & jnp.int32(
                    0xFF
                )
                plsc.addupdate_scatter(
                    hist_v, [lane_base + digit], ones16, mask=pm
                )

        side_hist(8, pre2)
        d2, kk, nn = pick_digit(kk, nn)
        pre3 = lax.shift_left(pre2, 8) | d2

        side_hist(0, pre3)
        d3, need_eq, _ = pick_digit(kk, nn)
        tkey = lax.shift_left(pre3, 8) | d3

        @plsc.parallel_loop(0, NVR, unroll=UNROLL, carry=zeros16)
        def off_final(i, off_vec):
            v = row_v[pl.ds(i * L, L)]
            key = _keys(v)
            ge = key >= tkey
            gei = ge.astype(jnp.int32)
            rank = plsc.cumsum(gei) - gei
            plsc.store_scatter(cmp_v, [off_vec + rank], v, mask=ge)
            return off_vec + _splat_pop(ge)

        plsc.store_scatter(cmp_v, [off_final + lane], neginf16)
        count_ge = jnp.max(off_final)

        nvr2 = (count_ge + jnp.int32(L)) // jnp.int32(L)

        @plsc.parallel_loop(0, nvr2, unroll=2, carry=(zeros16, zeros16))
        def _mfinal(i, carry):
            off_vec, eqc_vec = carry
            v = cmp_v[pl.ds(i * L, L)]
            key = _keys(v)
            eq = key == tkey
            gt = key > tkey
            eqi = eq.astype(jnp.int32)
            eqx = plsc.cumsum(eqi) - eqi
            sel = gt | (eq & ((eqc_vec + eqx) < need_eq))
            seli = sel.astype(jnp.int32)
            rank = plsc.cumsum(seli) - seli
            plsc.store_scatter(out_v, [off_vec + rank], v, mask=sel)
            return off_vec + _splat_pop(sel), eqc_vec + _splat_pop(eq)

        pltpu.sync_copy(out_v.at[pl.ds(0, K)], out_hbm.at[row])
        return c

    lax.fori_loop(0, ROWS_PER_W, do_row, 0)


_mesh = plsc.VectorSubcoreMesh(core_axis_name="c", subcore_axis_name="s")

_kmax_sc = functools.partial(
    pl.kernel,
    out_type=jax.ShapeDtypeStruct((R, K), jnp.float32),
    mesh=_mesh,
    scratch_types=[
        pltpu.VMEM((C,), jnp.float32),
        pltpu.VMEM((C + L,), jnp.float32),
        pltpu.VMEM((C + L,), jnp.float32),
        pltpu.VMEM((NB * L,), jnp.int32),
        pltpu.VMEM((NB,), jnp.int32),
        pltpu.VMEM((K + L,), jnp.float32),
    ],
    compiler_params=pltpu.CompilerParams(needs_layout_passes=False),
)(_kmax_body)


@jax.jit
def kernel(x):
    return _kmax_sc(x)


# speedup vs baseline: 21.7142x; 1.1502x over previous
import functools

import jax
import jax.numpy as jnp
from jax import lax
from jax.experimental import pallas as pl
from jax.experimental.pallas import tpu as pltpu
from jax.experimental.pallas import tpu_sc as plsc

R = 64
C = 32768
K = 512
L = 16
NVR = C // L
NB = 256
NW = 32
ROWS_PER_W = R // NW
UNROLL = 8


def _keys(v):
    b = lax.bitcast_convert_type(v + 0.0, jnp.int32)
    m = lax.shift_right_arithmetic(b, 31)
    return b ^ (m & jnp.int32(0x7FFFFFFF))


def _splat_pop(mask):
    return plsc.all_reduce_population_count(mask)


def _kmax_body(x_hbm, out_hbm, row_v, cmp_v, hist_v, tot_v, out_v):
    wid = lax.axis_index("s") * 2 + lax.axis_index("c")
    lane = lax.iota(jnp.int32, L)
    lane_base = lane * NB
    zeros16 = jnp.zeros((L,), jnp.int32)
    ones16 = jnp.ones((L,), jnp.int32)
    neginf16 = jnp.full((L,), -jnp.inf, jnp.float32)

    def zero_hist():
        @plsc.parallel_loop(0, NB, unroll=8)
        def _(i):
            hist_v[pl.ds(i * L, L)] = zeros16

    def pick_digit(kk, nn):
        nk = nn - kk

        def tbody(g, carry):
            cnt_vec, run_max, carry_p = carry
            tot = zeros16
            for l in range(L):
                tot = tot + hist_v[pl.ds(l * NB + g * L, L)]
            tot_v[pl.ds(g * L, L)] = tot
            p = plsc.cumsum(tot) + carry_p
            cond = p <= nk
            cnt_vec = cnt_vec + _splat_pop(cond)
            run_max = jnp.maximum(run_max, jnp.where(cond, p, 0))
            return cnt_vec, run_max, jnp.max(p)

        cnt_vec, run_max, _ = lax.fori_loop(
            0, NB // L, tbody, (zeros16, zeros16, jnp.int32(0))
        )
        dstar = jnp.max(cnt_vec)
        p_dm1 = jnp.max(run_max)

        def cbody(g, acc):
            tot = tot_v[pl.ds(g * L, L)]
            dv = g * L + lane
            return jnp.maximum(acc, jnp.where(dv == dstar, tot, 0))

        c_at = jnp.max(lax.fori_loop(0, NB // L, cbody, zeros16))
        kk_next = kk - (nn - (p_dm1 + c_at))
        return dstar, kk_next, c_at

    def do_row(r, c):
        row = wid * ROWS_PER_W + r
        pltpu.sync_copy(x_hbm.at[row], row_v)

        zero_hist()

        @plsc.parallel_loop(0, NVR, unroll=UNROLL)
        def _(i):
            key = _keys(row_v[pl.ds(i * L, L)])
            digit = (
                lax.shift_right_arithmetic(key, 24) & jnp.int32(0xFF)
            ) ^ jnp.int32(0x80)
            plsc.addupdate_scatter(hist_v, [lane_base + digit], ones16)

        d0, kk, nn = pick_digit(jnp.int32(K), jnp.int32(C))
        pre1 = lax.shift_right_arithmetic(
            lax.shift_left(d0 ^ jnp.int32(0x80), 24), 24
        )
        kfloor = lax.shift_left(pre1, 24)

        zero_hist()

        @plsc.parallel_loop(0, NVR, unroll=UNROLL, carry=zeros16)
        def ge_off(i, off_vec):
            v = row_v[pl.ds(i * L, L)]
            key = _keys(v)
            pm = lax.shift_right_arithmetic(key, 24) == pre1
            digit = lax.shift_right_arithmetic(key, 16) & jnp.int32(0xFF)
            plsc.addupdate_scatter(
                hist_v, [lane_base + digit], ones16, mask=pm
            )
            ge = key >= kfloor
            gei = ge.astype(jnp.int32)
            rank = plsc.cumsum(gei) - gei
            plsc.store_scatter(cmp_v, [off_vec + rank], v, mask=ge)
            return off_vec + _splat_pop(ge)

        plsc.store_scatter(cmp_v, [ge_off + lane], neginf16)
        n_ge = jnp.max(ge_off)
        nvr_g = (n_ge + jnp.int32(L)) // jnp.int32(L)
        d1, kk, nn = pick_digit(kk, nn)
        pre2 = lax.shift_left(pre1, 8) | d1

        def cand_hist(shift, prefix):
            zero_hist()

            @plsc.parallel_loop(0, nvr_g, unroll=2)
            def _(i):
                key = _keys(cmp_v[pl.ds(i * L, L)])
                pm = lax.shift_right_arithmetic(key, shift + 8) == prefix
                digit = lax.shift_right_arithmetic(key, shift) & jnp.int32(
                    0xFF
                )
                plsc.addupdate_scatter(
                    hist_v, [lane_base + digit], ones16, mask=pm
                )

        cand_hist(8, pre2)
        d2, kk, nn = pick_digit(kk, nn)
        pre3 = lax.shift_left(pre2, 8) | d2

        cand_hist(0, pre3)
        d3, need_eq, _ = pick_digit(kk, nn)
        tkey = lax.shift_left(pre3, 8) | d3

        @plsc.parallel_loop(0, nvr_g, unroll=2, carry=(zeros16, zeros16))
        def _mfinal(i, carry):
            off_vec, eqc_vec = carry
            v = cmp_v[pl.ds(i * L, L)]
            key = _keys(v)
            eq = key == tkey
            gt = key > tkey
            eqi = eq.astype(jnp.int32)
            eqx = plsc.cumsum(eqi) - eqi
            sel = gt | (eq & ((eqc_vec + eqx) < need_eq))
            seli = sel.astype(jnp.int32)
            rank = plsc.cumsum(seli) - seli
            plsc.store_scatter(out_v, [off_vec + rank], v, mask=sel)
            return off_vec + _splat_pop(sel), eqc_vec + _splat_pop(eq)

        pltpu.sync_copy(out_v.at[pl.ds(0, K)], out_hbm.at[row])
        return c

    lax.fori_loop(0, ROWS_PER_W, do_row, 0)


_mesh = plsc.VectorSubcoreMesh(core_axis_name="c", subcore_axis_name="s")

_kmax_sc = functools.partial(
    pl.kernel,
    out_type=jax.ShapeDtypeStruct((R, K), jnp.float32),
    mesh=_mesh,
    scratch_types=[
        pltpu.VMEM((C,), jnp.float32),
        pltpu.VMEM((C + L,), jnp.float32),
        pltpu.VMEM((NB * L,), jnp.int32),
        pltpu.VMEM((NB,), jnp.int32),
        pltpu.VMEM((K + L,), jnp.float32),
    ],
    compiler_params=pltpu.CompilerParams(needs_layout_passes=False),
)(_kmax_body)


@jax.jit
def kernel(x):
    return _kmax_sc(x)
